# baseline (speedup 1.0000x reference)
import functools

import jax
import jax.numpy as jnp
from jax import lax
from jax.experimental import pallas as pl
from jax.experimental.pallas import tpu as pltpu
from jax.experimental.pallas import tpu_sc as plsc

NUM_ENTITIES = 1000000
EMBED_DIM = 32
BATCH = 16384

NC = 2
NS = 16
NW = NC * NS
BPW = BATCH // NW
IDX_CHUNK = 128
N_IDX_CHUNKS = BPW // IDX_CHUNK
LANES = 16


def _neg_sqrt(s):
    i = lax.bitcast_convert_type(s, jnp.int32)
    i = jnp.full((LANES,), 0x5F3759DF, jnp.int32) - (i >> 1)
    r = lax.bitcast_convert_type(i, jnp.float32)
    for _ in range(3):
        r = r * (1.5 - 0.5 * s * r * r)
    return -(s * r)


def _sc_body(batch_hbm, table_hbm, out_hbm,
             batch_v, idx_h, idx_t, rows_h, rows_t, out_v, sem):
    wid = lax.axis_index("s") * NC + lax.axis_index("c")
    base = wid * BPW

    pltpu.sync_copy(batch_hbm.at[pl.ds(base * 3, BPW * 3)], batch_v)

    iota = lax.iota(jnp.int32, LANES)
    for j in range(N_IDX_CHUNKS):
        for c in range(IDX_CHUNK // LANES):
            ri = ((j * IDX_CHUNK + c * LANES) + iota) * 3
            idx_h[j, pl.ds(c * LANES, LANES)] = plsc.load_gather(
                batch_v, [ri])
            idx_t[j, pl.ds(c * LANES, LANES)] = plsc.load_gather(
                batch_v, [ri + 2])

    copies = []
    for j in range(N_IDX_CHUNKS):
        copies.append(pltpu.make_async_copy(
            table_hbm.at[idx_h.at[j]],
            rows_h.at[pl.ds(j * IDX_CHUNK, IDX_CHUNK)], sem))
        copies.append(pltpu.make_async_copy(
            table_hbm.at[idx_t.at[j]],
            rows_t.at[pl.ds(j * IDX_CHUNK, IDX_CHUNK)], sem))
    for cp in copies:
        cp.start()
    for cp in copies:
        cp.wait()

    def chunk_body(c, carry):
        ri = c * LANES + iota
        acc = jnp.zeros((LANES,), jnp.float32)
        for d in range(EMBED_DIM):
            dsplat = jnp.full((LANES,), d, jnp.int32)
            hv = plsc.load_gather(rows_h, [ri, dsplat])
            tv = plsc.load_gather(rows_t, [ri, dsplat])
            df = hv - tv
            acc = acc + df * df
        out_v[pl.ds(c * LANES, LANES)] = _neg_sqrt(acc + 1e-12)
        return carry

    lax.fori_loop(0, BPW // LANES, chunk_body, 0)

    pltpu.sync_copy(out_v, out_hbm.at[pl.ds(base, BPW)])


@functools.partial(jax.jit, static_argnames=())
def _sc_score(batch, entity_embeddings):
    mesh = plsc.VectorSubcoreMesh(core_axis_name="c", subcore_axis_name="s")
    return pl.kernel(
        _sc_body,
        out_type=jax.ShapeDtypeStruct((BATCH,), jnp.float32),
        mesh=mesh,
        compiler_params=pltpu.CompilerParams(
            needs_layout_passes=False, use_tc_tiling_on_sc=False),
        scratch_types=[
            pltpu.VMEM((BPW * 3,), jnp.int32),
            pltpu.VMEM((N_IDX_CHUNKS, IDX_CHUNK), jnp.int32),
            pltpu.VMEM((N_IDX_CHUNKS, IDX_CHUNK), jnp.int32),
            pltpu.VMEM((BPW, EMBED_DIM), jnp.float32),
            pltpu.VMEM((BPW, EMBED_DIM), jnp.float32),
            pltpu.VMEM((BPW,), jnp.float32),
            pltpu.SemaphoreType.DMA,
        ],
    )(batch.reshape(-1), entity_embeddings)


def kernel(batch, entity_embeddings):
    return _sc_score(batch, entity_embeddings)
